# baseline (speedup 1.0000x reference)
import jax
import jax.numpy as jnp
from jax import lax
from jax.experimental import pallas as pl
from jax.experimental.pallas import tpu as pltpu
from jax.experimental.pallas import tpu_sc as plsc

BATCH = 16384
EMBED_DIM = 32

_NC = 2
_NS = 16
_NW = _NC * _NS
_BPW = BATCH // _NW
_CHUNK = 128
_NCHUNK = _BPW // _CHUNK


def _sc_lookup(labels2d, drop_i32, table, null_emb):
    mesh = plsc.VectorSubcoreMesh(core_axis_name="c", subcore_axis_name="s")

    @pl.kernel(
        mesh=mesh,
        compiler_params=pltpu.CompilerParams(
            needs_layout_passes=False, use_tc_tiling_on_sc=False),
        out_type=jax.ShapeDtypeStruct((BATCH, EMBED_DIM), jnp.float32),
        scratch_types=[
            pltpu.VMEM((_NCHUNK, _CHUNK), jnp.int32),
            pltpu.VMEM((_BPW,), jnp.int32),
            pltpu.VMEM((EMBED_DIM,), jnp.float32),
            pltpu.VMEM((_BPW, EMBED_DIM), jnp.float32),
            pltpu.SemaphoreType.DMA,
        ],
    )
    def k(labels_hbm, drop_hbm, table_hbm, null_hbm, out_hbm,
          idx_v, drop_v, null_v, rows_v, sem):
        wid = lax.axis_index("s") * _NC + lax.axis_index("c")
        base = wid * _BPW

        pltpu.sync_copy(labels_hbm.at[pl.ds(wid * _NCHUNK, _NCHUNK)], idx_v)
        pltpu.sync_copy(drop_hbm.at[pl.ds(base, _BPW)], drop_v)
        pltpu.sync_copy(null_hbm, null_v)

        copies = [
            pltpu.async_copy(
                table_hbm.at[idx_v.at[j]],
                rows_v.at[pl.ds(j * _CHUNK, _CHUNK)],
                sem,
            )
            for j in range(_NCHUNK)
        ]
        for cp in copies:
            cp.wait()

        iota = lax.iota(jnp.int32, 16)
        null_splat = [
            plsc.load_gather(null_v, [jnp.full((16,), c, jnp.int32)])
            for c in range(EMBED_DIM)
        ]

        def group_body(g, carry):
            drop16 = drop_v[pl.ds(g * 16, 16)]
            m = drop16 != 0
            rows16 = g * 16 + iota
            for c in range(EMBED_DIM):
                plsc.store_scatter(
                    rows_v, [rows16, jnp.full((16,), c, jnp.int32)],
                    null_splat[c], mask=m)
            return carry

        lax.fori_loop(0, _BPW // 16, group_body, 0)

        pltpu.sync_copy(rows_v, out_hbm.at[pl.ds(base, _BPW)])

    return k(labels2d, drop_i32, table, null_emb)


def kernel(labels, force_drop_ids, table, null_embedding):
    labels2d = labels.astype(jnp.int32).reshape(BATCH // _CHUNK, _CHUNK)
    drop_i32 = force_drop_ids.astype(jnp.int32)
    return _sc_lookup(labels2d, drop_i32, table, null_embedding)
